# baseline (speedup 1.0000x reference)
import functools

import jax
import jax.numpy as jnp
from jax import lax
from jax.experimental import pallas as pl
from jax.experimental.pallas import tpu as pltpu
from jax.experimental.pallas import tpu_sc as plsc

NUM_EMB = 1000000
DIM = 64
B, T = 4096, 50
TOTAL = B * T
NC, NS = 2, 16
NW = NC * NS
PER_W = TOTAL // NW
CHUNK = 128
N_CHUNKS = PER_W // CHUNK


def _sc_gather(idx3d, weight):
  mesh = plsc.VectorSubcoreMesh(core_axis_name="c", subcore_axis_name="s")

  @functools.partial(
      pl.kernel,
      mesh=mesh,
      out_type=jax.ShapeDtypeStruct((TOTAL, DIM), jnp.float32),
      compiler_params=pltpu.CompilerParams(use_tc_tiling_on_sc=False),
      scratch_types=[
          pltpu.VMEM((N_CHUNKS, CHUNK), jnp.int32),
          pltpu.VMEM((CHUNK, DIM), jnp.float32),
          pltpu.VMEM((CHUNK, DIM), jnp.float32),
          pltpu.SemaphoreType.DMA,
          pltpu.SemaphoreType.DMA,
      ],
  )
  def k(idx_hbm, table_hbm, out_hbm, idx_v, rows_a, rows_b, sem_a, sem_b):
    wid = lax.axis_index("s") * NC + lax.axis_index("c")
    base = wid * PER_W
    pltpu.sync_copy(idx_hbm.at[wid], idx_v)

    bufs = ((rows_a, sem_a), (rows_b, sem_b))
    pltpu.async_copy(table_hbm.at[idx_v.at[0]], rows_a, sem_a)
    pltpu.async_copy(table_hbm.at[idx_v.at[1]], rows_b, sem_b)

    def body(g, carry):
      for p, (buf, sem) in enumerate(bufs):
        j = 2 * g + p
        pltpu.make_async_copy(table_hbm.at[idx_v.at[0]], buf, sem).wait()
        pltpu.sync_copy(buf, out_hbm.at[pl.ds(base + j * CHUNK, CHUNK)])

        @pl.when(j + 2 < N_CHUNKS)
        def _():
          pltpu.async_copy(table_hbm.at[idx_v.at[j + 2]], buf, sem)

      return carry

    lax.fori_loop(0, N_CHUNKS // 2, body, 0)

  return k(idx3d, weight)


def kernel(IX, weight):
  idx3d = IX.reshape(NW, N_CHUNKS, CHUNK).astype(jnp.int32)
  out = _sc_gather(idx3d, weight)
  return out.reshape(B, T, DIM)


# speedup vs baseline: 1.0086x; 1.0086x over previous
import functools

import jax
import jax.numpy as jnp
from jax import lax
from jax.experimental import pallas as pl
from jax.experimental.pallas import tpu as pltpu
from jax.experimental.pallas import tpu_sc as plsc

NUM_EMB = 1000000
DIM = 64
B, T = 4096, 50
TOTAL = B * T
NC, NS = 2, 16
NW = NC * NS
PER_W = TOTAL // NW
CHUNK = 128
N_CHUNKS = PER_W // CHUNK
NBUF = 5


def _sc_gather(idx3d, weight):
  mesh = plsc.VectorSubcoreMesh(core_axis_name="c", subcore_axis_name="s")

  @functools.partial(
      pl.kernel,
      mesh=mesh,
      out_type=jax.ShapeDtypeStruct((TOTAL, DIM), jnp.float32),
      compiler_params=pltpu.CompilerParams(use_tc_tiling_on_sc=False),
      scratch_types=[
          pltpu.VMEM((N_CHUNKS, CHUNK), jnp.int32),
      ] + [pltpu.VMEM((CHUNK, DIM), jnp.float32) for _ in range(NBUF)]
        + [pltpu.SemaphoreType.DMA for _ in range(NBUF)],
  )
  def k(idx_hbm, table_hbm, out_hbm, idx_v, *bufs_and_sems):
    rows = bufs_and_sems[:NBUF]
    sems = bufs_and_sems[NBUF:]
    wid = lax.axis_index("s") * NC + lax.axis_index("c")
    base = wid * PER_W
    pltpu.sync_copy(idx_hbm.at[wid], idx_v)

    for p in range(NBUF):
      pltpu.async_copy(table_hbm.at[idx_v.at[p]], rows[p], sems[p])

    def body(g, carry):
      for p in range(NBUF):
        j = NBUF * g + p
        buf, sem = rows[p], sems[p]
        pltpu.make_async_copy(table_hbm.at[idx_v.at[0]], buf, sem).wait()
        pltpu.sync_copy(buf, out_hbm.at[pl.ds(base + j * CHUNK, CHUNK)])

        @pl.when(j + NBUF < N_CHUNKS)
        def _():
          pltpu.async_copy(table_hbm.at[idx_v.at[j + NBUF]], buf, sem)

      return carry

    lax.fori_loop(0, N_CHUNKS // NBUF, body, 0)

  return k(idx3d, weight)


def kernel(IX, weight):
  idx3d = IX.reshape(NW, N_CHUNKS, CHUNK).astype(jnp.int32)
  out = _sc_gather(idx3d, weight)
  return out.reshape(B, T, DIM)


# speedup vs baseline: 1.5649x; 1.5515x over previous
import functools

import jax
import jax.numpy as jnp
from jax import lax
from jax.experimental import pallas as pl
from jax.experimental.pallas import tpu as pltpu
from jax.experimental.pallas import tpu_sc as plsc

NUM_EMB = 1000000
DIM = 64
B, T = 4096, 50
TOTAL = B * T
NC, NS = 2, 16
NW = NC * NS
PER_W = TOTAL // NW
CHUNK = 128
N_CHUNKS = PER_W // CHUNK
NBUF = 5


def _sc_gather(idx3d, weight):
  mesh = plsc.VectorSubcoreMesh(core_axis_name="c", subcore_axis_name="s")

  @functools.partial(
      pl.kernel,
      mesh=mesh,
      out_type=jax.ShapeDtypeStruct((TOTAL, DIM), jnp.float32),
      compiler_params=pltpu.CompilerParams(use_tc_tiling_on_sc=False),
      scratch_types=[
          pltpu.VMEM((N_CHUNKS, CHUNK), jnp.int32),
      ] + [pltpu.VMEM((CHUNK, DIM), jnp.float32) for _ in range(NBUF)]
        + [pltpu.SemaphoreType.DMA for _ in range(NBUF)],
  )
  def k(idx_hbm, table_hbm, out_hbm, idx_v, *bufs_and_sems):
    rows = bufs_and_sems[:NBUF]
    sems = bufs_and_sems[NBUF:]
    wid = lax.axis_index("s") * NC + lax.axis_index("c")
    base = wid * PER_W
    pltpu.sync_copy(idx_hbm.at[wid], idx_v)

    for p in range(NBUF):
      pltpu.async_copy(table_hbm.at[idx_v.at[p]], rows[p], sems[p])

    def body(g, carry):
      for p in range(NBUF):
        j = NBUF * g + p
        buf, sem = rows[p], sems[p]
        pltpu.make_async_copy(table_hbm.at[idx_v.at[0]], buf, sem).wait()
        pltpu.sync_copy(buf, out_hbm.at[pl.ds(base + j * CHUNK, CHUNK)])

        @pl.when(j + NBUF < N_CHUNKS)
        def _():
          pltpu.async_copy(table_hbm.at[idx_v.at[j + NBUF]], buf, sem)

      return carry

    lax.fori_loop(0, N_CHUNKS // NBUF, body, 0)

  return k(idx3d, weight)


TN = 2048


def _tc_transpose(wt):
  def body(in_ref, out_ref):
    x = in_ref[...]
    out_ref[:, :DIM] = jnp.transpose(x[:, :TN // 2], (1, 0))
    out_ref[:, DIM:] = jnp.transpose(x[:, TN // 2:], (1, 0))

  grid = (NUM_EMB + TN - 1) // TN
  n_rows = grid * (TN // 2)
  return pl.pallas_call(
      body,
      grid=(grid,),
      in_specs=[pl.BlockSpec((DIM, TN), lambda i: (0, i))],
      out_specs=pl.BlockSpec((TN // 2, 2 * DIM), lambda i: (i, 0)),
      out_shape=jax.ShapeDtypeStruct((n_rows, 2 * DIM), jnp.float32),
  )(wt)


def kernel(IX, weight):
  wt_pairs = _tc_transpose(weight.T)
  wt_rows = wt_pairs.reshape(-1, DIM)
  ix = IX.astype(jnp.int32)
  c = ix % TN
  ixm = (ix // TN) * TN + 2 * (c % (TN // 2)) + c // (TN // 2)
  idx3d = ixm.reshape(NW, N_CHUNKS, CHUNK)
  out = _sc_gather(idx3d, wt_rows)
  return out.reshape(B, T, DIM)
